# baseline (speedup 1.0000x reference)
import functools

import jax
import jax.numpy as jnp
from jax import lax
from jax.experimental import pallas as pl
from jax.experimental.pallas import tpu as pltpu
from jax.experimental.pallas import tpu_sc as plsc

N = 5000
TILE = 1024
T = 5
NPAD = 5120
MAX_DETECTION = 1000
DET_THRESHOLD = 0.2
IOU_THRESHOLD = 0.5
F = 6


def _nms_body(coords, x0r, y0r, x1r, y1r, sr, sup_ref, rank_ref):
    xr0 = [x0r[pl.ds(t * 8, 8), :] for t in range(T)]
    yr0 = [y0r[pl.ds(t * 8, 8), :] for t in range(T)]
    xr1 = [x1r[pl.ds(t * 8, 8), :] for t in range(T)]
    yr1 = [y1r[pl.ds(t * 8, 8), :] for t in range(T)]
    sj = [sr[pl.ds(t * 8, 8), :] for t in range(T)]
    area_r = [(xr1[t] - xr0[t]) * (yr1[t] - yr0[t]) for t in range(T)]

    zero = xr0[0] * 0.0
    one = zero + 1.0

    def group(i, t, mode, jlin):
        base = i * F
        x0 = coords[base]
        y0 = coords[base + 1]
        x1 = coords[base + 2]
        y1 = coords[base + 3]
        ai = coords[base + 4]
        si = coords[base + 5]
        ltx = jnp.maximum(xr0[t], x0)
        lty = jnp.maximum(yr0[t], y0)
        rbx = jnp.minimum(xr1[t], x1)
        rby = jnp.minimum(yr1[t], y1)
        w = jnp.maximum(rbx - ltx, 0.0)
        h = jnp.maximum(rby - lty, 0.0)
        inter = w * h
        union = (ai + area_r[t]) - inter
        if mode == 0:
            prec = si >= sj[t]
        elif mode == 1:
            prec = (si > sj[t]) | ((si == sj[t]) & (i < jlin))
        else:
            prec = si > sj[t]
        p01 = jnp.where(prec, 1.0, 0.0)
        return inter * p01, union, p01

    def finish(pend, acc):
        pi, pu = pend
        return jnp.maximum(acc, pi / pu)

    def make_step(r, jlin):
        def step(i, st):
            accs, rks, pend = st
            na, nr, np_ = [], [], []
            for t in range(T):
                mode = 1 if t == r else (2 if t < r else 0)
                acc = finish(pend[t], accs[t])
                interp, union, p01 = group(i, t, mode, jlin)
                np_.append((interp, union))
                na.append(acc)
                nr.append(rks[t] + p01)
            return tuple(na), tuple(nr), tuple(np_)
        return step

    st = ((zero,) * T, (zero,) * T,
          tuple((zero, one) for _ in range(T)))
    iota2d = (jax.lax.broadcasted_iota(jnp.int32, (8, 128), 0) * 128
              + jax.lax.broadcasted_iota(jnp.int32, (8, 128), 1))
    for r in range(T):
        lo, hi = TILE * r, min(TILE * (r + 1), N)
        jlin = iota2d + TILE * r
        st = jax.lax.fori_loop(lo, hi, make_step(r, jlin), st)
    accs, rks, pend = st
    for t in range(T):
        acc = finish(pend[t], accs[t])
        sup_ref[pl.ds(t * 8, 8), :] = acc
        rank_ref[pl.ds(t * 8, 8), :] = rks[t]


def _nms_pass(coords_smem, rows):
    smem_spec = pl.BlockSpec(memory_space=pltpu.SMEM)
    return pl.pallas_call(
        _nms_body,
        in_specs=[smem_spec]
        + [pl.BlockSpec((NPAD // 128, 128), lambda: (0, 0))] * 5,
        out_specs=[pl.BlockSpec((NPAD // 128, 128), lambda: (0, 0))] * 2,
        out_shape=[jax.ShapeDtypeStruct((NPAD // 128, 128), jnp.float32)] * 2,
    )(coords_smem, *rows)


KPAD = 1024


def _sc_gather_boxes(bp, idx):
    mesh = plsc.VectorSubcoreMesh(core_axis_name="c", subcore_axis_name="s")
    per_w = KPAD // 32

    @functools.partial(
        pl.kernel,
        mesh=mesh,
        out_type=jax.ShapeDtypeStruct((KPAD, 4), jnp.float32),
        scratch_types=[
            pltpu.VMEM((per_w,), jnp.int32),
            pltpu.VMEM((per_w, 4), jnp.float32),
            pltpu.SemaphoreType.DMA,
        ],
        compiler_params=pltpu.CompilerParams(use_tc_tiling_on_sc=False),
    )
    def gk(bh, idxh, o, idx_v, rows_v, sem):
        wid = lax.axis_index("s") * 2 + lax.axis_index("c")
        base = wid * per_w
        pltpu.sync_copy(idxh.at[pl.ds(base, per_w)], idx_v)
        pltpu.async_copy(bh.at[idx_v], rows_v, sem).wait()
        pltpu.sync_copy(rows_v, o.at[pl.ds(base, per_w)])

    return gk(bp, idx)


def kernel(boxes, scores):
    area = (boxes[:, 2] - boxes[:, 0]) * (boxes[:, 3] - boxes[:, 1])
    coords = jnp.concatenate(
        [boxes, area[:, None], scores[:, None]], axis=1).reshape(-1)
    bp = jnp.pad(boxes, ((0, NPAD - N), (0, 0)))
    sp = jnp.pad(scores, (0, NPAD - N))
    rows = [bp[:, k].reshape(NPAD // 128, 128) for k in range(4)]
    rows.append(sp.reshape(NPAD // 128, 128))
    sup, rank = _nms_pass(coords, rows)
    sup = sup.reshape(NPAD)[:N]
    rank = rank.reshape(NPAD)[:N]
    keep = (sup <= IOU_THRESHOLD) & (scores >= DET_THRESHOLD)
    key = jnp.where(keep, scores, -(rank + 1.0))
    top_key, top_idx = jax.lax.top_k(key, MAX_DETECTION)
    top_scores = jnp.maximum(top_key, 0.0)
    idx_pad = jnp.pad(top_idx, (0, KPAD - MAX_DETECTION))
    top_boxes = _sc_gather_boxes(bp, idx_pad)[:MAX_DETECTION]
    return jnp.concatenate([top_boxes, top_scores[:, None]], axis=1)


# speedup vs baseline: 1.1617x; 1.1617x over previous
import jax
import jax.numpy as jnp
from jax.experimental import pallas as pl
from jax.experimental.pallas import tpu as pltpu

N = 5000
TILE = 1024
T = 5
NPAD = 5120
MAX_DETECTION = 1000
DET_THRESHOLD = 0.2
IOU_THRESHOLD = 0.5
F = 6


def _nms_body(coords, x0r, y0r, x1r, y1r, sr, sup_ref, rank_ref):
    xr0 = [x0r[pl.ds(t * 8, 8), :] for t in range(T)]
    yr0 = [y0r[pl.ds(t * 8, 8), :] for t in range(T)]
    xr1 = [x1r[pl.ds(t * 8, 8), :] for t in range(T)]
    yr1 = [y1r[pl.ds(t * 8, 8), :] for t in range(T)]
    sj = [sr[pl.ds(t * 8, 8), :] for t in range(T)]
    area_r = [(xr1[t] - xr0[t]) * (yr1[t] - yr0[t]) for t in range(T)]

    zero = xr0[0] * 0.0
    one = zero + 1.0

    def group(i, t, mode, jlin):
        base = i * F
        x0 = coords[base]
        y0 = coords[base + 1]
        x1 = coords[base + 2]
        y1 = coords[base + 3]
        ai = coords[base + 4]
        si = coords[base + 5]
        ltx = jnp.maximum(xr0[t], x0)
        lty = jnp.maximum(yr0[t], y0)
        rbx = jnp.minimum(xr1[t], x1)
        rby = jnp.minimum(yr1[t], y1)
        w = jnp.maximum(rbx - ltx, 0.0)
        h = jnp.maximum(rby - lty, 0.0)
        inter = w * h
        union = (ai + area_r[t]) - inter
        if mode == 0:
            prec = si >= sj[t]
        elif mode == 1:
            prec = (si > sj[t]) | ((si == sj[t]) & (i < jlin))
        else:
            prec = si > sj[t]
        p01 = jnp.where(prec, 1.0, 0.0)
        return inter * p01, union, p01

    def finish(pend, acc):
        pi, pu = pend
        return jnp.maximum(acc, pi / pu)

    def make_step(r, jlin):
        def step(i, st):
            accs, rks, pend = st
            na, nr, np_ = [], [], []
            for t in range(T):
                mode = 1 if t == r else (2 if t < r else 0)
                acc = finish(pend[t], accs[t])
                interp, union, p01 = group(i, t, mode, jlin)
                np_.append((interp, union))
                na.append(acc)
                nr.append(rks[t] + p01)
            return tuple(na), tuple(nr), tuple(np_)
        return step

    st = ((zero,) * T, (zero,) * T,
          tuple((zero, one) for _ in range(T)))
    iota2d = (jax.lax.broadcasted_iota(jnp.int32, (8, 128), 0) * 128
              + jax.lax.broadcasted_iota(jnp.int32, (8, 128), 1))
    for r in range(T):
        lo, hi = TILE * r, min(TILE * (r + 1), N)
        jlin = iota2d + TILE * r
        st = jax.lax.fori_loop(lo, hi, make_step(r, jlin), st)
    accs, rks, pend = st
    for t in range(T):
        acc = finish(pend[t], accs[t])
        sup_ref[pl.ds(t * 8, 8), :] = acc
        rank_ref[pl.ds(t * 8, 8), :] = rks[t]


def _nms_pass(coords_smem, rows):
    smem_spec = pl.BlockSpec(memory_space=pltpu.SMEM)
    return pl.pallas_call(
        _nms_body,
        in_specs=[smem_spec]
        + [pl.BlockSpec((NPAD // 128, 128), lambda: (0, 0))] * 5,
        out_specs=[pl.BlockSpec((NPAD // 128, 128), lambda: (0, 0))] * 2,
        out_shape=[jax.ShapeDtypeStruct((NPAD // 128, 128), jnp.float32)] * 2,
    )(coords_smem, *rows)


def kernel(boxes, scores):
    area = (boxes[:, 2] - boxes[:, 0]) * (boxes[:, 3] - boxes[:, 1])
    coords = jnp.concatenate(
        [boxes, area[:, None], scores[:, None]], axis=1).reshape(-1)
    bp = jnp.pad(boxes, ((0, NPAD - N), (0, 0)))
    sp = jnp.pad(scores, (0, NPAD - N))
    rows = [bp[:, k].reshape(NPAD // 128, 128) for k in range(4)]
    rows.append(sp.reshape(NPAD // 128, 128))
    sup, rank = _nms_pass(coords, rows)
    sup = sup.reshape(NPAD)[:N]
    rank = rank.reshape(NPAD)[:N]
    keep = (sup <= IOU_THRESHOLD) & (scores >= DET_THRESHOLD)
    key = jnp.where(keep, scores, -(rank + 1.0))
    top_key, top_idx = jax.lax.top_k(key, MAX_DETECTION)
    top_scores = jnp.maximum(top_key, 0.0)
    top_boxes = jnp.take(boxes, top_idx, axis=0)
    return jnp.concatenate([top_boxes, top_scores[:, None]], axis=1)


# speedup vs baseline: 1.1628x; 1.0009x over previous
import jax
import jax.numpy as jnp
from jax.experimental import pallas as pl
from jax.experimental.pallas import tpu as pltpu

N = 5000
TILE = 1024
T = 5
NPAD = 5120
MAX_DETECTION = 1000
DET_THRESHOLD = 0.2
IOU_THRESHOLD = 0.5
F = 6


def _nms_body(coords, x0r, y0r, x1r, y1r, sr, sup_ref, rank_ref):
    xr0 = [x0r[pl.ds(t * 8, 8), :] for t in range(T)]
    yr0 = [y0r[pl.ds(t * 8, 8), :] for t in range(T)]
    xr1 = [x1r[pl.ds(t * 8, 8), :] for t in range(T)]
    yr1 = [y1r[pl.ds(t * 8, 8), :] for t in range(T)]
    sj = [sr[pl.ds(t * 8, 8), :] for t in range(T)]
    area_r = [(xr1[t] - xr0[t]) * (yr1[t] - yr0[t]) for t in range(T)]

    zero = xr0[0] * 0.0
    one = zero + 1.0

    def group(i, t, mode, jlin):
        base = i * F
        x0 = coords[base]
        y0 = coords[base + 1]
        x1 = coords[base + 2]
        y1 = coords[base + 3]
        ai = coords[base + 4]
        si = coords[base + 5]
        ltx = jnp.maximum(xr0[t], x0)
        lty = jnp.maximum(yr0[t], y0)
        rbx = jnp.minimum(xr1[t], x1)
        rby = jnp.minimum(yr1[t], y1)
        w = jnp.maximum(rbx - ltx, 0.0)
        h = jnp.maximum(rby - lty, 0.0)
        inter = w * h
        union = (ai + area_r[t]) - inter
        if mode == 0:
            prec = si >= sj[t]
        elif mode == 1:
            prec = (si > sj[t]) | ((si == sj[t]) & (i < jlin))
        else:
            prec = si > sj[t]
        p01 = jnp.where(prec, 1.0, 0.0)
        return inter, jnp.where(prec, union, -1.0), p01

    def finish(pend, acc):
        pi, pu = pend
        return jnp.maximum(acc, pi / pu)

    def make_step(r, jlin):
        def step(i, st):
            accs, rks, pend = st
            na, nr, np_ = [], [], []
            for t in range(T):
                mode = 1 if t == r else (2 if t < r else 0)
                acc = finish(pend[t], accs[t])
                interp, union, p01 = group(i, t, mode, jlin)
                np_.append((interp, union))
                na.append(acc)
                nr.append(rks[t] + p01)
            return tuple(na), tuple(nr), tuple(np_)
        return step

    st = ((zero,) * T, (zero,) * T,
          tuple((zero, one) for _ in range(T)))
    iota2d = (jax.lax.broadcasted_iota(jnp.int32, (8, 128), 0) * 128
              + jax.lax.broadcasted_iota(jnp.int32, (8, 128), 1))
    for r in range(T):
        lo, hi = TILE * r, min(TILE * (r + 1), N)
        jlin = iota2d + TILE * r
        st = jax.lax.fori_loop(lo, hi, make_step(r, jlin), st)
    accs, rks, pend = st
    for t in range(T):
        acc = finish(pend[t], accs[t])
        sup_ref[pl.ds(t * 8, 8), :] = acc
        rank_ref[pl.ds(t * 8, 8), :] = rks[t]


def _nms_pass(coords_smem, rows):
    smem_spec = pl.BlockSpec(memory_space=pltpu.SMEM)
    return pl.pallas_call(
        _nms_body,
        in_specs=[smem_spec]
        + [pl.BlockSpec((NPAD // 128, 128), lambda: (0, 0))] * 5,
        out_specs=[pl.BlockSpec((NPAD // 128, 128), lambda: (0, 0))] * 2,
        out_shape=[jax.ShapeDtypeStruct((NPAD // 128, 128), jnp.float32)] * 2,
    )(coords_smem, *rows)


def kernel(boxes, scores):
    area = (boxes[:, 2] - boxes[:, 0]) * (boxes[:, 3] - boxes[:, 1])
    coords = jnp.concatenate(
        [boxes, area[:, None], scores[:, None]], axis=1).reshape(-1)
    bp = jnp.pad(boxes, ((0, NPAD - N), (0, 0)))
    sp = jnp.pad(scores, (0, NPAD - N))
    rows = [bp[:, k].reshape(NPAD // 128, 128) for k in range(4)]
    rows.append(sp.reshape(NPAD // 128, 128))
    sup, rank = _nms_pass(coords, rows)
    sup = sup.reshape(NPAD)[:N]
    rank = rank.reshape(NPAD)[:N]
    keep = (sup <= IOU_THRESHOLD) & (scores >= DET_THRESHOLD)
    key = jnp.where(keep, scores, -(rank + 1.0))
    top_key, top_idx = jax.lax.top_k(key, MAX_DETECTION)
    top_scores = jnp.maximum(top_key, 0.0)
    top_boxes = jnp.take(boxes, top_idx, axis=0)
    return jnp.concatenate([top_boxes, top_scores[:, None]], axis=1)


# speedup vs baseline: 1.3678x; 1.1763x over previous
import jax
import jax.numpy as jnp
from jax.experimental import pallas as pl
from jax.experimental.pallas import tpu as pltpu

N = 5000
TILE = 1024
T = 5
NPAD = 5120
MAX_DETECTION = 1000
DET_THRESHOLD = 0.2
IOU_THRESHOLD = 0.5
F = 6


def _nms_body(coords, x0r, y0r, x1r, y1r, sr, sup_ref, rank_ref):
    xr0 = [x0r[pl.ds(t * 8, 8), :] for t in range(T)]
    yr0 = [y0r[pl.ds(t * 8, 8), :] for t in range(T)]
    xr1 = [x1r[pl.ds(t * 8, 8), :] for t in range(T)]
    yr1 = [y1r[pl.ds(t * 8, 8), :] for t in range(T)]
    sj = [sr[pl.ds(t * 8, 8), :] for t in range(T)]
    area_r = [(xr1[t] - xr0[t]) * (yr1[t] - yr0[t]) for t in range(T)]

    zero = xr0[0] * 0.0
    one = zero + 1.0

    def group(i, t, mode, jlin):
        base = i * F
        x0 = coords[base]
        y0 = coords[base + 1]
        x1 = coords[base + 2]
        y1 = coords[base + 3]
        ai = coords[base + 4]
        si = coords[base + 5]
        ltx = jnp.maximum(xr0[t], x0)
        lty = jnp.maximum(yr0[t], y0)
        rbx = jnp.minimum(xr1[t], x1)
        rby = jnp.minimum(yr1[t], y1)
        w = jnp.maximum(rbx - ltx, 0.0)
        h = jnp.maximum(rby - lty, 0.0)
        inter = w * h
        union = (ai + area_r[t]) - inter
        if mode == 0:
            prec = si >= sj[t]
        elif mode == 1:
            prec = (si > sj[t]) | ((si == sj[t]) & (i < jlin))
        else:
            prec = si > sj[t]
        p01 = jnp.where(prec, 1.0, 0.0)
        return inter, jnp.where(prec, union, -1.0), p01

    def finish(pend, acc):
        pi, pu = pend
        return jnp.maximum(acc, pi / pu)

    def make_step(r, jlin):
        def step(s, st):
            accs, rks, pend = st
            na, nr, np_ = [], [], []
            modes = [1 if t == r else (2 if t < r else 0) for t in range(T)]
            for t in range(T):
                acc = finish(pend[t], accs[t])
                g = group(2 * s, t, modes[t], jlin)
                na.append(finish((g[0], g[1]), acc))
                nr.append(rks[t] + g[2])
            for t in range(T):
                interp, union, p01 = group(2 * s + 1, t, modes[t], jlin)
                np_.append((interp, union))
                nr[t] = nr[t] + p01
            return tuple(na), tuple(nr), tuple(np_)
        return step

    st = ((zero,) * T, (zero,) * T,
          tuple((zero, one) for _ in range(T)))
    iota2d = (jax.lax.broadcasted_iota(jnp.int32, (8, 128), 0) * 128
              + jax.lax.broadcasted_iota(jnp.int32, (8, 128), 1))
    for r in range(T):
        lo, hi = TILE * r, min(TILE * (r + 1), N)
        jlin = iota2d + TILE * r
        st = jax.lax.fori_loop(lo // 2, hi // 2, make_step(r, jlin), st)
    accs, rks, pend = st
    for t in range(T):
        acc = finish(pend[t], accs[t])
        sup_ref[pl.ds(t * 8, 8), :] = acc
        rank_ref[pl.ds(t * 8, 8), :] = rks[t]


def _nms_pass(coords_smem, rows):
    smem_spec = pl.BlockSpec(memory_space=pltpu.SMEM)
    return pl.pallas_call(
        _nms_body,
        in_specs=[smem_spec]
        + [pl.BlockSpec((NPAD // 128, 128), lambda: (0, 0))] * 5,
        out_specs=[pl.BlockSpec((NPAD // 128, 128), lambda: (0, 0))] * 2,
        out_shape=[jax.ShapeDtypeStruct((NPAD // 128, 128), jnp.float32)] * 2,
    )(coords_smem, *rows)


def kernel(boxes, scores):
    area = (boxes[:, 2] - boxes[:, 0]) * (boxes[:, 3] - boxes[:, 1])
    coords = jnp.concatenate(
        [boxes, area[:, None], scores[:, None]], axis=1).reshape(-1)
    bp = jnp.pad(boxes, ((0, NPAD - N), (0, 0)))
    sp = jnp.pad(scores, (0, NPAD - N))
    rows = [bp[:, k].reshape(NPAD // 128, 128) for k in range(4)]
    rows.append(sp.reshape(NPAD // 128, 128))
    sup, rank = _nms_pass(coords, rows)
    sup = sup.reshape(NPAD)[:N]
    rank = rank.reshape(NPAD)[:N]
    keep = (sup <= IOU_THRESHOLD) & (scores >= DET_THRESHOLD)
    key = jnp.where(keep, scores, -(rank + 1.0))
    top_key, top_idx = jax.lax.top_k(key, MAX_DETECTION)
    top_scores = jnp.maximum(top_key, 0.0)
    top_boxes = jnp.take(boxes, top_idx, axis=0)
    return jnp.concatenate([top_boxes, top_scores[:, None]], axis=1)


# speedup vs baseline: 1.5267x; 1.1162x over previous
import jax
import jax.numpy as jnp
from jax.experimental import pallas as pl
from jax.experimental.pallas import tpu as pltpu

N = 5000
TILE = 1024
T = 5
NPAD = 5120
MAX_DETECTION = 1000
DET_THRESHOLD = 0.2
IOU_THRESHOLD = 0.5
F = 6
UI = 4


def _nms_body(coords, x0r, y0r, x1r, y1r, sr, sup_ref, rank_ref):
    xr0 = [x0r[pl.ds(t * 8, 8), :] for t in range(T)]
    yr0 = [y0r[pl.ds(t * 8, 8), :] for t in range(T)]
    xr1 = [x1r[pl.ds(t * 8, 8), :] for t in range(T)]
    yr1 = [y1r[pl.ds(t * 8, 8), :] for t in range(T)]
    sj = [sr[pl.ds(t * 8, 8), :] for t in range(T)]
    area_r = [(xr1[t] - xr0[t]) * (yr1[t] - yr0[t]) for t in range(T)]

    zero = xr0[0] * 0.0
    one = zero + 1.0

    def group(i, t, mode, jlin):
        base = i * F
        x0 = coords[base]
        y0 = coords[base + 1]
        x1 = coords[base + 2]
        y1 = coords[base + 3]
        ai = coords[base + 4]
        si = coords[base + 5]
        ltx = jnp.maximum(xr0[t], x0)
        lty = jnp.maximum(yr0[t], y0)
        rbx = jnp.minimum(xr1[t], x1)
        rby = jnp.minimum(yr1[t], y1)
        w = jnp.maximum(rbx - ltx, 0.0)
        h = jnp.maximum(rby - lty, 0.0)
        inter = w * h
        union = (ai + area_r[t]) - inter
        if mode == 0:
            prec = si >= sj[t]
        elif mode == 1:
            prec = (si > sj[t]) | ((si == sj[t]) & (i < jlin))
        else:
            prec = si > sj[t]
        p01 = jnp.where(prec, 1.0, 0.0)
        return inter, jnp.where(prec, union, -1.0), p01

    def finish(pend, acc):
        pi, pu = pend
        return jnp.maximum(acc, pi / pu)

    def make_step(r, jlin):
        def step(s, st):
            accs, rks, pend = st
            na, nr, np_ = [], [], []
            modes = [1 if t == r else (2 if t < r else 0) for t in range(T)]
            for t in range(T):
                na.append(finish(pend[t], accs[t]))
                nr.append(rks[t])
            for u in range(UI - 1):
                for t in range(T):
                    g = group(UI * s + u, t, modes[t], jlin)
                    na[t] = finish((g[0], g[1]), na[t])
                    nr[t] = nr[t] + g[2]
            for t in range(T):
                interp, union, p01 = group(UI * s + UI - 1, t, modes[t], jlin)
                np_.append((interp, union))
                nr[t] = nr[t] + p01
            return tuple(na), tuple(nr), tuple(np_)
        return step

    st = ((zero,) * T, (zero,) * T,
          tuple((zero, one) for _ in range(T)))
    iota2d = (jax.lax.broadcasted_iota(jnp.int32, (8, 128), 0) * 128
              + jax.lax.broadcasted_iota(jnp.int32, (8, 128), 1))
    for r in range(T):
        lo, hi = TILE * r, min(TILE * (r + 1), N)
        jlin = iota2d + TILE * r
        st = jax.lax.fori_loop(lo // UI, hi // UI, make_step(r, jlin), st)
    accs, rks, pend = st
    for t in range(T):
        acc = finish(pend[t], accs[t])
        sup_ref[pl.ds(t * 8, 8), :] = acc
        rank_ref[pl.ds(t * 8, 8), :] = rks[t]


def _nms_pass(coords_smem, rows):
    smem_spec = pl.BlockSpec(memory_space=pltpu.SMEM)
    return pl.pallas_call(
        _nms_body,
        in_specs=[smem_spec]
        + [pl.BlockSpec((NPAD // 128, 128), lambda: (0, 0))] * 5,
        out_specs=[pl.BlockSpec((NPAD // 128, 128), lambda: (0, 0))] * 2,
        out_shape=[jax.ShapeDtypeStruct((NPAD // 128, 128), jnp.float32)] * 2,
    )(coords_smem, *rows)


def kernel(boxes, scores):
    area = (boxes[:, 2] - boxes[:, 0]) * (boxes[:, 3] - boxes[:, 1])
    coords = jnp.concatenate(
        [boxes, area[:, None], scores[:, None]], axis=1).reshape(-1)
    bp = jnp.pad(boxes, ((0, NPAD - N), (0, 0)))
    sp = jnp.pad(scores, (0, NPAD - N))
    rows = [bp[:, k].reshape(NPAD // 128, 128) for k in range(4)]
    rows.append(sp.reshape(NPAD // 128, 128))
    sup, rank = _nms_pass(coords, rows)
    sup = sup.reshape(NPAD)[:N]
    rank = rank.reshape(NPAD)[:N]
    keep = (sup <= IOU_THRESHOLD) & (scores >= DET_THRESHOLD)
    key = jnp.where(keep, scores, -(rank + 1.0))
    top_key, top_idx = jax.lax.top_k(key, MAX_DETECTION)
    top_scores = jnp.maximum(top_key, 0.0)
    top_boxes = jnp.take(boxes, top_idx, axis=0)
    return jnp.concatenate([top_boxes, top_scores[:, None]], axis=1)


# speedup vs baseline: 1.6113x; 1.0554x over previous
import jax
import jax.numpy as jnp
from jax.experimental import pallas as pl
from jax.experimental.pallas import tpu as pltpu

N = 5000
TILE = 1024
T = 5
NPAD = 5120
MAX_DETECTION = 1000
DET_THRESHOLD = 0.2
IOU_THRESHOLD = 0.5
F = 6
UI = 8


def _nms_body(coords, x0r, y0r, x1r, y1r, sr, sup_ref, rank_ref):
    xr0 = [x0r[pl.ds(t * 8, 8), :] for t in range(T)]
    yr0 = [y0r[pl.ds(t * 8, 8), :] for t in range(T)]
    xr1 = [x1r[pl.ds(t * 8, 8), :] for t in range(T)]
    yr1 = [y1r[pl.ds(t * 8, 8), :] for t in range(T)]
    sj = [sr[pl.ds(t * 8, 8), :] for t in range(T)]
    area_r = [(xr1[t] - xr0[t]) * (yr1[t] - yr0[t]) for t in range(T)]

    zero = xr0[0] * 0.0
    one = zero + 1.0

    def group(i, t, mode, jlin):
        base = i * F
        x0 = coords[base]
        y0 = coords[base + 1]
        x1 = coords[base + 2]
        y1 = coords[base + 3]
        ai = coords[base + 4]
        si = coords[base + 5]
        ltx = jnp.maximum(xr0[t], x0)
        lty = jnp.maximum(yr0[t], y0)
        rbx = jnp.minimum(xr1[t], x1)
        rby = jnp.minimum(yr1[t], y1)
        w = jnp.maximum(rbx - ltx, 0.0)
        h = jnp.maximum(rby - lty, 0.0)
        inter = w * h
        union = (ai + area_r[t]) - inter
        if mode == 0:
            prec = si >= sj[t]
        elif mode == 1:
            prec = (si > sj[t]) | ((si == sj[t]) & (i < jlin))
        else:
            prec = si > sj[t]
        p01 = jnp.where(prec, 1.0, 0.0)
        return inter, jnp.where(prec, union, -1.0), p01

    def finish(pend, acc):
        pi, pu = pend
        return jnp.maximum(acc, pi / pu)

    def make_step(r, jlin):
        def step(s, st):
            accs, rks, pend = st
            na, nr, np_ = [], [], []
            modes = [1 if t == r else (2 if t < r else 0) for t in range(T)]
            for t in range(T):
                na.append(finish(pend[t], accs[t]))
                nr.append(rks[t])
            for u in range(UI - 1):
                for t in range(T):
                    g = group(UI * s + u, t, modes[t], jlin)
                    na[t] = finish((g[0], g[1]), na[t])
                    nr[t] = nr[t] + g[2]
            for t in range(T):
                interp, union, p01 = group(UI * s + UI - 1, t, modes[t], jlin)
                np_.append((interp, union))
                nr[t] = nr[t] + p01
            return tuple(na), tuple(nr), tuple(np_)
        return step

    st = ((zero,) * T, (zero,) * T,
          tuple((zero, one) for _ in range(T)))
    iota2d = (jax.lax.broadcasted_iota(jnp.int32, (8, 128), 0) * 128
              + jax.lax.broadcasted_iota(jnp.int32, (8, 128), 1))
    for r in range(T):
        lo, hi = TILE * r, min(TILE * (r + 1), N)
        jlin = iota2d + TILE * r
        st = jax.lax.fori_loop(lo // UI, hi // UI, make_step(r, jlin), st)
    accs, rks, pend = st
    for t in range(T):
        acc = finish(pend[t], accs[t])
        sup_ref[pl.ds(t * 8, 8), :] = acc
        rank_ref[pl.ds(t * 8, 8), :] = rks[t]


def _nms_pass(coords_smem, rows):
    smem_spec = pl.BlockSpec(memory_space=pltpu.SMEM)
    return pl.pallas_call(
        _nms_body,
        in_specs=[smem_spec]
        + [pl.BlockSpec((NPAD // 128, 128), lambda: (0, 0))] * 5,
        out_specs=[pl.BlockSpec((NPAD // 128, 128), lambda: (0, 0))] * 2,
        out_shape=[jax.ShapeDtypeStruct((NPAD // 128, 128), jnp.float32)] * 2,
    )(coords_smem, *rows)


def kernel(boxes, scores):
    area = (boxes[:, 2] - boxes[:, 0]) * (boxes[:, 3] - boxes[:, 1])
    coords = jnp.concatenate(
        [boxes, area[:, None], scores[:, None]], axis=1).reshape(-1)
    bp = jnp.pad(boxes, ((0, NPAD - N), (0, 0)))
    sp = jnp.pad(scores, (0, NPAD - N))
    rows = [bp[:, k].reshape(NPAD // 128, 128) for k in range(4)]
    rows.append(sp.reshape(NPAD // 128, 128))
    sup, rank = _nms_pass(coords, rows)
    sup = sup.reshape(NPAD)[:N]
    rank = rank.reshape(NPAD)[:N]
    keep = (sup <= IOU_THRESHOLD) & (scores >= DET_THRESHOLD)
    key = jnp.where(keep, scores, -(rank + 1.0))
    top_key, top_idx = jax.lax.top_k(key, MAX_DETECTION)
    top_scores = jnp.maximum(top_key, 0.0)
    top_boxes = jnp.take(boxes, top_idx, axis=0)
    return jnp.concatenate([top_boxes, top_scores[:, None]], axis=1)


# speedup vs baseline: 1.6663x; 1.0342x over previous
import jax
import jax.numpy as jnp
from jax.experimental import pallas as pl
from jax.experimental.pallas import tpu as pltpu

N = 5000
TILE = 1024
T = 5
NPAD = 5120
MAX_DETECTION = 1000
DET_THRESHOLD = 0.2
IOU_THRESHOLD = 0.5
F = 6
UI = 16


def _nms_body(coords, x0r, y0r, x1r, y1r, sr, sup_ref, rank_ref):
    xr0 = [x0r[pl.ds(t * 8, 8), :] for t in range(T)]
    yr0 = [y0r[pl.ds(t * 8, 8), :] for t in range(T)]
    xr1 = [x1r[pl.ds(t * 8, 8), :] for t in range(T)]
    yr1 = [y1r[pl.ds(t * 8, 8), :] for t in range(T)]
    sj = [sr[pl.ds(t * 8, 8), :] for t in range(T)]
    area_r = [(xr1[t] - xr0[t]) * (yr1[t] - yr0[t]) for t in range(T)]

    zero = xr0[0] * 0.0
    one = zero + 1.0

    def group(i, t, mode, jlin):
        base = i * F
        x0 = coords[base]
        y0 = coords[base + 1]
        x1 = coords[base + 2]
        y1 = coords[base + 3]
        ai = coords[base + 4]
        si = coords[base + 5]
        ltx = jnp.maximum(xr0[t], x0)
        lty = jnp.maximum(yr0[t], y0)
        rbx = jnp.minimum(xr1[t], x1)
        rby = jnp.minimum(yr1[t], y1)
        w = jnp.maximum(rbx - ltx, 0.0)
        h = jnp.maximum(rby - lty, 0.0)
        inter = w * h
        union = (ai + area_r[t]) - inter
        if mode == 0:
            prec = si >= sj[t]
        elif mode == 1:
            prec = (si > sj[t]) | ((si == sj[t]) & (i < jlin))
        else:
            prec = si > sj[t]
        p01 = jnp.where(prec, 1.0, 0.0)
        return inter, jnp.where(prec, union, -1.0), p01

    def finish(pend, acc):
        pi, pu = pend
        return jnp.maximum(acc, pi / pu)

    def make_step(r, jlin):
        def step(s, st):
            accs, rks, pend = st
            na, nr, np_ = [], [], []
            modes = [1 if t == r else (2 if t < r else 0) for t in range(T)]
            for t in range(T):
                na.append(finish(pend[t], accs[t]))
                nr.append(rks[t])
            for u in range(UI - 1):
                for t in range(T):
                    g = group(UI * s + u, t, modes[t], jlin)
                    na[t] = finish((g[0], g[1]), na[t])
                    nr[t] = nr[t] + g[2]
            for t in range(T):
                interp, union, p01 = group(UI * s + UI - 1, t, modes[t], jlin)
                np_.append((interp, union))
                nr[t] = nr[t] + p01
            return tuple(na), tuple(nr), tuple(np_)
        return step

    st = ((zero,) * T, (zero,) * T,
          tuple((zero, one) for _ in range(T)))
    iota2d = (jax.lax.broadcasted_iota(jnp.int32, (8, 128), 0) * 128
              + jax.lax.broadcasted_iota(jnp.int32, (8, 128), 1))
    for r in range(T):
        lo, hi = TILE * r, min(TILE * (r + 1), N)
        jlin = iota2d + TILE * r
        nfull = (hi - lo) // UI
        st = jax.lax.fori_loop(lo // UI, lo // UI + nfull, make_step(r, jlin), st)
        for i in range(lo + nfull * UI, hi):
            accs, rks, pend = st
            mode = [1 if t == r else (2 if t < r else 0) for t in range(T)]
            na, nr, np_ = [], [], []
            for t in range(T):
                acc = finish(pend[t], accs[t])
                interp, union, p01 = group(i, t, mode[t], jlin)
                np_.append((interp, union))
                na.append(acc)
                nr.append(rks[t] + p01)
            st = (tuple(na), tuple(nr), tuple(np_))
    accs, rks, pend = st
    for t in range(T):
        acc = finish(pend[t], accs[t])
        sup_ref[pl.ds(t * 8, 8), :] = acc
        rank_ref[pl.ds(t * 8, 8), :] = rks[t]


def _nms_pass(coords_smem, rows):
    smem_spec = pl.BlockSpec(memory_space=pltpu.SMEM)
    return pl.pallas_call(
        _nms_body,
        in_specs=[smem_spec]
        + [pl.BlockSpec((NPAD // 128, 128), lambda: (0, 0))] * 5,
        out_specs=[pl.BlockSpec((NPAD // 128, 128), lambda: (0, 0))] * 2,
        out_shape=[jax.ShapeDtypeStruct((NPAD // 128, 128), jnp.float32)] * 2,
    )(coords_smem, *rows)


def kernel(boxes, scores):
    area = (boxes[:, 2] - boxes[:, 0]) * (boxes[:, 3] - boxes[:, 1])
    coords = jnp.concatenate(
        [boxes, area[:, None], scores[:, None]], axis=1).reshape(-1)
    bp = jnp.pad(boxes, ((0, NPAD - N), (0, 0)))
    sp = jnp.pad(scores, (0, NPAD - N))
    rows = [bp[:, k].reshape(NPAD // 128, 128) for k in range(4)]
    rows.append(sp.reshape(NPAD // 128, 128))
    sup, rank = _nms_pass(coords, rows)
    sup = sup.reshape(NPAD)[:N]
    rank = rank.reshape(NPAD)[:N]
    keep = (sup <= IOU_THRESHOLD) & (scores >= DET_THRESHOLD)
    key = jnp.where(keep, scores, -(rank + 1.0))
    top_key, top_idx = jax.lax.top_k(key, MAX_DETECTION)
    top_scores = jnp.maximum(top_key, 0.0)
    top_boxes = jnp.take(boxes, top_idx, axis=0)
    return jnp.concatenate([top_boxes, top_scores[:, None]], axis=1)
